# baseline (speedup 1.0000x reference)
import jax
import jax.numpy as jnp
from jax.experimental import pallas as pl
from jax.experimental.pallas import tpu as pltpu

N = 4096
N_IN = 512
N_H = 128

TM = 512
NSTEPS = N // (2 * TM)
HALF_T = N // 2


def _mega_kernel(adja_ref, adjb_ref, s1_ref, s2_ref, w_ref, b_ref,
                 h1a_ref, h1b_ref, h2a_ref, h2b_ref, acc_ref, f_ref):
    i = pl.program_id(0)

    @pl.when(i == 0)
    def _():
        w = w_ref[...]
        f_ref[:, :N_H] = jnp.dot(s1_ref[...], w, preferred_element_type=jnp.float32)
        f_ref[:, N_H:] = jnp.dot(s2_ref[...], w, preferred_element_type=jnp.float32)

    f = f_ref[...]
    b = b_ref[...]
    agg_a = jnp.dot(adja_ref[...], f, preferred_element_type=jnp.float32)
    agg_b = jnp.dot(adjb_ref[...], f, preferred_element_type=jnp.float32)
    h1a = jnp.maximum(agg_a[:, :N_H] + b, 0.0)
    h2a = jnp.maximum(agg_a[:, N_H:] + b, 0.0)
    h1b = jnp.maximum(agg_b[:, :N_H] + b, 0.0)
    h2b = jnp.maximum(agg_b[:, N_H:] + b, 0.0)
    h1a_ref[...] = h1a.astype(jnp.bfloat16)
    h2a_ref[...] = h2a.astype(jnp.bfloat16)
    h1b_ref[...] = h1b.astype(jnp.bfloat16)
    h2b_ref[...] = h2b.astype(jnp.bfloat16)
    part = (jnp.sum(h1a, axis=0, keepdims=True)
            + jnp.sum(h1b, axis=0, keepdims=True))

    @pl.when(i == 0)
    def _():
        acc_ref[...] = part

    @pl.when(i != 0)
    def _():
        acc_ref[...] += part


def _fin_kernel(h1a_ref, h1b_ref, h2a_ref, h2b_ref, acc_ref, wb_ref,
                prompt_ref, bb_ref, o_ref):
    c = jax.nn.sigmoid(acc_ref[...] * (1.0 / N))
    v = jax.lax.dot_general(c, wb_ref[...], (((1,), (1,)), ((), ())),
                            preferred_element_type=jnp.float32)
    v2 = v * prompt_ref[...]
    bb = bb_ref[0, 0]
    dn = (((1,), (1,)), ((), ()))
    h1a = h1a_ref[...].astype(jnp.float32)
    h1b = h1b_ref[...].astype(jnp.float32)
    h2a = h2a_ref[...].astype(jnp.float32)
    h2b = h2b_ref[...].astype(jnp.float32)
    o_ref[0:1, :HALF_T] = jax.lax.dot_general(v, h1a, dn, preferred_element_type=jnp.float32) + bb
    o_ref[0:1, HALF_T:] = jax.lax.dot_general(v, h1b, dn, preferred_element_type=jnp.float32) + bb
    o_ref[1:2, :HALF_T] = jax.lax.dot_general(v2, h2a, dn, preferred_element_type=jnp.float32) + bb
    o_ref[1:2, HALF_T:] = jax.lax.dot_general(v2, h2b, dn, preferred_element_type=jnp.float32) + bb


def kernel(seq1, seq2, adj, sparse, W_gcn, b_gcn, prompt, W_bil, b_bil):
    s1 = seq1[0]
    s2 = seq2[0]
    a = adj[0]
    b2 = b_gcn.reshape(1, N_H)
    bb = b_bil.reshape(1, 1)
    off = HALF_T // TM

    h1a, h1b, h2a, h2b, acc = pl.pallas_call(
        _mega_kernel,
        grid=(NSTEPS,),
        in_specs=[
            pl.BlockSpec((TM, N), lambda i: (i, 0)),
            pl.BlockSpec((TM, N), lambda i: (i + off, 0)),
            pl.BlockSpec((N, N_IN), lambda i: (0, 0)),
            pl.BlockSpec((N, N_IN), lambda i: (0, 0)),
            pl.BlockSpec((N_IN, N_H), lambda i: (0, 0)),
            pl.BlockSpec((1, N_H), lambda i: (0, 0)),
        ],
        out_specs=[
            pl.BlockSpec((TM, N_H), lambda i: (i, 0)),
            pl.BlockSpec((TM, N_H), lambda i: (i, 0)),
            pl.BlockSpec((TM, N_H), lambda i: (i, 0)),
            pl.BlockSpec((TM, N_H), lambda i: (i, 0)),
            pl.BlockSpec((1, N_H), lambda i: (0, 0)),
        ],
        out_shape=[
            jax.ShapeDtypeStruct((HALF_T, N_H), jnp.bfloat16),
            jax.ShapeDtypeStruct((HALF_T, N_H), jnp.bfloat16),
            jax.ShapeDtypeStruct((HALF_T, N_H), jnp.bfloat16),
            jax.ShapeDtypeStruct((HALF_T, N_H), jnp.bfloat16),
            jax.ShapeDtypeStruct((1, N_H), jnp.float32),
        ],
        scratch_shapes=[pltpu.VMEM((N, 2 * N_H), jnp.float32)],
    )(a, a, s1, s2, W_gcn, b2)

    out = pl.pallas_call(
        _fin_kernel,
        in_specs=[
            pl.BlockSpec((HALF_T, N_H), lambda: (0, 0)),
            pl.BlockSpec((HALF_T, N_H), lambda: (0, 0)),
            pl.BlockSpec((HALF_T, N_H), lambda: (0, 0)),
            pl.BlockSpec((HALF_T, N_H), lambda: (0, 0)),
            pl.BlockSpec((1, N_H), lambda: (0, 0)),
            pl.BlockSpec((N_H, N_H), lambda: (0, 0)),
            pl.BlockSpec((1, N_H), lambda: (0, 0)),
            pl.BlockSpec((1, 1), lambda: (0, 0)),
        ],
        out_specs=pl.BlockSpec((2, N), lambda: (0, 0)),
        out_shape=jax.ShapeDtypeStruct((2, N), jnp.float32),
    )(h1a, h1b, h2a, h2b, acc, W_bil, prompt, bb)

    return out.reshape(1, 2 * N)


# speedup vs baseline: 1.0892x; 1.0892x over previous
import jax
import jax.numpy as jnp
from jax.experimental import pallas as pl
from jax.experimental.pallas import tpu as pltpu

N = 4096
N_IN = 512
N_H = 128

TM = 512
NSTEPS = N // TM


def _dgi_kernel(adj_ref, s1_ref, s2_ref, w_ref, b_ref, wb_ref, prompt_ref,
                bb_ref, o_ref, f_ref, h_ref, acc_ref):
    i = pl.program_id(0)

    @pl.when(i == 0)
    def _():
        w = w_ref[...]
        f_ref[:, :N_H] = jnp.dot(s1_ref[...], w, preferred_element_type=jnp.float32)
        f_ref[:, N_H:] = jnp.dot(s2_ref[...], w, preferred_element_type=jnp.float32)

    agg = jnp.dot(adj_ref[...], f_ref[...], preferred_element_type=jnp.float32)
    b = b_ref[...]
    h1 = jnp.maximum(agg[:, :N_H] + b, 0.0)
    h2 = jnp.maximum(agg[:, N_H:] + b, 0.0)
    h_ref[pl.ds(i * TM, TM), :N_H] = h1
    h_ref[pl.ds(i * TM, TM), N_H:] = h2
    part = jnp.sum(h1, axis=0, keepdims=True)

    @pl.when(i == 0)
    def _():
        acc_ref[...] = part

    @pl.when(i != 0)
    def _():
        acc_ref[...] += part

    @pl.when(i == NSTEPS - 1)
    def _():
        c = jax.nn.sigmoid(acc_ref[...] * (1.0 / N))
        v = jax.lax.dot_general(c, wb_ref[...], (((1,), (1,)), ((), ())),
                                preferred_element_type=jnp.float32)
        v2 = v * prompt_ref[...]
        bb = bb_ref[0, 0]
        dn = (((1,), (1,)), ((), ()))
        o_ref[0:1, :] = jax.lax.dot_general(
            v, h_ref[:, :N_H], dn, preferred_element_type=jnp.float32) + bb
        o_ref[1:2, :] = jax.lax.dot_general(
            v2, h_ref[:, N_H:], dn, preferred_element_type=jnp.float32) + bb


def kernel(seq1, seq2, adj, sparse, W_gcn, b_gcn, prompt, W_bil, b_bil):
    s1 = seq1[0]
    s2 = seq2[0]
    a = adj[0]
    b2 = b_gcn.reshape(1, N_H)
    bb = b_bil.reshape(1, 1)

    out = pl.pallas_call(
        _dgi_kernel,
        grid=(NSTEPS,),
        in_specs=[
            pl.BlockSpec((TM, N), lambda i: (i, 0)),
            pl.BlockSpec((N, N_IN), lambda i: (0, 0)),
            pl.BlockSpec((N, N_IN), lambda i: (0, 0)),
            pl.BlockSpec((N_IN, N_H), lambda i: (0, 0)),
            pl.BlockSpec((1, N_H), lambda i: (0, 0)),
            pl.BlockSpec((N_H, N_H), lambda i: (0, 0)),
            pl.BlockSpec((1, N_H), lambda i: (0, 0)),
            pl.BlockSpec((1, 1), lambda i: (0, 0)),
        ],
        out_specs=pl.BlockSpec((2, N), lambda i: (0, 0)),
        out_shape=jax.ShapeDtypeStruct((2, N), jnp.float32),
        scratch_shapes=[
            pltpu.VMEM((N, 2 * N_H), jnp.float32),
            pltpu.VMEM((N, 2 * N_H), jnp.float32),
            pltpu.VMEM((1, N_H), jnp.float32),
        ],
    )(a, s1, s2, W_gcn, b2, W_bil, prompt, bb)

    return out.reshape(1, 2 * N)


# speedup vs baseline: 1.1186x; 1.0270x over previous
import jax
import jax.numpy as jnp
from jax.experimental import pallas as pl
from jax.experimental.pallas import tpu as pltpu

N = 4096
N_IN = 512
N_H = 128

TM = 1024
NSTEPS = N // TM


def _dgi_kernel(adj_ref, s1_ref, s2_ref, w_ref, b_ref, wb_ref, prompt_ref,
                bb_ref, o_ref, f_ref, h_ref, acc_ref):
    i = pl.program_id(0)

    @pl.when(i == 0)
    def _():
        w = w_ref[...]
        f_ref[:, :N_H] = jnp.dot(s1_ref[...], w, preferred_element_type=jnp.float32)
        f_ref[:, N_H:] = jnp.dot(s2_ref[...], w, preferred_element_type=jnp.float32)

    agg = jnp.dot(adj_ref[...], f_ref[...], preferred_element_type=jnp.float32)
    b = b_ref[...]
    h1 = jnp.maximum(agg[:, :N_H] + b, 0.0)
    h2 = jnp.maximum(agg[:, N_H:] + b, 0.0)
    h_ref[pl.ds(i * TM, TM), :N_H] = h1
    h_ref[pl.ds(i * TM, TM), N_H:] = h2
    part = jnp.sum(h1, axis=0, keepdims=True)

    @pl.when(i == 0)
    def _():
        acc_ref[...] = part

    @pl.when(i != 0)
    def _():
        acc_ref[...] += part

    @pl.when(i == NSTEPS - 1)
    def _():
        c = jax.nn.sigmoid(acc_ref[...] * (1.0 / N))
        v = jax.lax.dot_general(c, wb_ref[...], (((1,), (1,)), ((), ())),
                                preferred_element_type=jnp.float32)
        v2 = v * prompt_ref[...]
        bb = bb_ref[0, 0]
        dn = (((1,), (1,)), ((), ()))
        o_ref[0:1, :] = jax.lax.dot_general(
            v, h_ref[:, :N_H], dn, preferred_element_type=jnp.float32) + bb
        o_ref[1:2, :] = jax.lax.dot_general(
            v2, h_ref[:, N_H:], dn, preferred_element_type=jnp.float32) + bb


def kernel(seq1, seq2, adj, sparse, W_gcn, b_gcn, prompt, W_bil, b_bil):
    s1 = seq1[0]
    s2 = seq2[0]
    a = adj[0]
    b2 = b_gcn.reshape(1, N_H)
    bb = b_bil.reshape(1, 1)

    out = pl.pallas_call(
        _dgi_kernel,
        grid=(NSTEPS,),
        in_specs=[
            pl.BlockSpec((TM, N), lambda i: (i, 0)),
            pl.BlockSpec((N, N_IN), lambda i: (0, 0)),
            pl.BlockSpec((N, N_IN), lambda i: (0, 0)),
            pl.BlockSpec((N_IN, N_H), lambda i: (0, 0)),
            pl.BlockSpec((1, N_H), lambda i: (0, 0)),
            pl.BlockSpec((N_H, N_H), lambda i: (0, 0)),
            pl.BlockSpec((1, N_H), lambda i: (0, 0)),
            pl.BlockSpec((1, 1), lambda i: (0, 0)),
        ],
        out_specs=pl.BlockSpec((2, N), lambda i: (0, 0)),
        out_shape=jax.ShapeDtypeStruct((2, N), jnp.float32),
        scratch_shapes=[
            pltpu.VMEM((N, 2 * N_H), jnp.float32),
            pltpu.VMEM((N, 2 * N_H), jnp.float32),
            pltpu.VMEM((1, N_H), jnp.float32),
        ],
        compiler_params=pltpu.CompilerParams(
            vmem_limit_bytes=100 * 1024 * 1024),
    )(a, s1, s2, W_gcn, b2, W_bil, prompt, bb)

    return out.reshape(1, 2 * N)
